# Initial kernel scaffold; baseline (speedup 1.0000x reference)
import jax
import jax.numpy as jnp
from jax.experimental import pallas as pl


def kernel(x, zero_map, DBC, f, c11_w_1, c11_b_1, c12_w_1, c12_b_1, c13_w_1, c13_b_1, convt_w_1, c11_w_3, c11_b_3, c12_w_3, c12_b_3, c13_w_3, c13_b_3, convt_w_3, c11_w_5, c11_b_5, c12_w_5, c12_b_5, c13_w_5, c13_b_5, convt_w_5):
    raise NotImplementedError("write your pallas kernel here")


# speedup vs baseline: 1.1925x; 1.1925x over previous
import functools
import numpy as np
import jax
import jax.numpy as jnp
from jax import lax
from jax.experimental import pallas as pl
from jax.experimental.pallas import tpu as pltpu
from jax.experimental.pallas import tpu_sc as plsc

RES = 23
NE = RES * RES
NDOF = 1152
WROW = 1168
NT = 16
TR = NDOF // NT
BATCH = 16
KS = (1, 3, 5)
CH = {1: 64, 3: 64, 5: 32}
LP = {1: 16, 3: 32, 5: 80}
VW = 128
NC, NS = 2, 16
NW = NC * NS


def _build_static():
    out = {}
    for k in KS:
        kk = k + 1
        L = 2 * kk * kk
        n2 = 2 * (RES + k) * (RES + k)
        glob = np.arange((RES + k) * (RES + k)).reshape(RES + k, RES + k)
        loc = np.zeros((NE, L), dtype=np.int64)
        for i in range(NE):
            col = i % RES
            row = i // RES
            gv = glob[row:row + kk, col:col + kk].flatten()
            loc[i] = np.vstack([2 * gv, 2 * gv + 1]).T.flatten()
        if k == 1:
            fmap = np.arange(n2)
        else:
            pad = (k - 1) // 2
            inner_node = glob[pad:-pad, pad:-pad].flatten()
            inner = np.vstack([2 * inner_node, 2 * inner_node + 1]).T.flatten()
            fmap = -np.ones(n2, dtype=np.int64)
            fmap[inner] = np.arange(NDOF)
        floc = fmap[loc]
        lists = [[] for _ in range(NT)]
        for e in range(NE):
            fr = floc[e]
            for p in range(L):
                r = fr[p]
                if r < 0:
                    continue
                t = r // TR
                lists[t].append((e * L + p, (r - t * TR) * WROW, e))
        ch = CH[k]
        lp = LP[k]
        npad = ((max(len(l) for l in lists) + ch - 1) // ch) * ch
        gid = np.zeros((NT, npad), dtype=np.int32)
        addr = np.zeros((NT, npad, lp), dtype=np.int32)
        trash_base = TR * WROW
        lanes = np.arange(lp)
        for t in range(NT):
            for i in range(npad):
                if i < len(lists[t]):
                    g, base, e = lists[t][i]
                    gid[t, i] = g
                    cols = np.concatenate([floc[e], -np.ones(lp - L, np.int64)])
                    addr[t, i] = np.where(
                        cols >= 0, base + cols, base + NDOF + (lanes % 16))
                else:
                    gid[t, i] = 0
                    addr[t, i] = trash_base + (lanes % 16)
        out[k] = (gid, addr, npad, L, lp, ch)
    return out


_WL = _build_static()


def _tri_offsets(L):
    return [i * (i + 1) // 2 for i in range(L)]


def _diag_mask(L):
    ltri = L * (L + 1) // 2
    m = np.zeros((1, ltri), dtype=np.float32)
    for i in range(L):
        m[0, i * (i + 1) // 2 + i] = 1.0
    return m


NB = 23
NPB = NE // NB


def _tc_body(*refs):
    ins = refs[:27]
    o = {1: refs[27], 3: refs[28], 5: refs[29]}
    b3s = {1: refs[30], 3: refs[31], 5: refs[32]}
    nb = pl.program_id(1)
    for j, k in enumerate(KS):
        pat, w1, b1, w2, b2, w3, b3, wm, dm = ins[9 * j:9 * (j + 1)]
        L = 2 * (k + 1) ** 2
        b3r = b3s[k]

        @pl.when(nb == 0)
        def _(pat=pat, w1=w1, b1=b1, w2=w2, b2=b2, w3=w3, b3=b3, wm=wm,
              dm=dm, b3r=b3r, L=L):
            p = pat[0]
            h = jnp.maximum(jnp.dot(p, w1[...], preferred_element_type=jnp.float32) + b1[...], 0.0)
            h = jnp.maximum(jnp.dot(h, w2[...], preferred_element_type=jnp.float32) + b2[...], 0.0)
            h = jnp.maximum(jnp.dot(h, w3[...], preferred_element_type=jnp.float32) + b3[...], 0.0)
            bf = jnp.dot(h, wm[...], preferred_element_type=jnp.float32)
            bf = jnp.where(dm[...] > 0.5, jnp.maximum(bf, 0.0), bf)
            offs = _tri_offsets(L)
            b3r[...] = jnp.zeros((NE, L, L), jnp.float32)
            for i in range(L):
                b3r[:, i, 0:i + 1] = bf[:, offs[i]:offs[i] + i + 1]

        oref = o[k]
        oref[0, :, L:VW] = jnp.zeros((NPB * L, VW - L), jnp.float32)

        def mm(m, _, b3r=b3r, oref=oref, L=L, nb=nb):
            a = b3r[pl.ds(nb * NPB + m, 1)][0]
            r = lax.dot_general(a, a, (((1,), (1,)), ((), ())),
                                preferred_element_type=jnp.float32)
            oref[0, pl.ds(m * L, L), 0:L] = r
            return 0

        lax.fori_loop(0, NPB, mm, 0)


def _run_tc(pats, weights):
    in_specs = []
    args = []
    for k in KS:
        L = 2 * (k + 1) ** 2
        ck = 2 * k * k
        w1, b1, w2, b2, w3, b3, wm = weights[k]
        dm = jnp.asarray(_diag_mask(L))
        args += [pats[k], w1, b1, w2, b2, w3, b3, wm, dm]
        in_specs += [
            pl.BlockSpec((1, NE, ck), lambda b, nb: (b, 0, 0)),
            pl.BlockSpec(w1.shape, lambda b, nb: (0, 0)),
            pl.BlockSpec(b1.shape, lambda b, nb: (0, 0)),
            pl.BlockSpec(w2.shape, lambda b, nb: (0, 0)),
            pl.BlockSpec(b2.shape, lambda b, nb: (0, 0)),
            pl.BlockSpec(w3.shape, lambda b, nb: (0, 0)),
            pl.BlockSpec(b3.shape, lambda b, nb: (0, 0)),
            pl.BlockSpec(wm.shape, lambda b, nb: (0, 0)),
            pl.BlockSpec(dm.shape, lambda b, nb: (0, 0)),
        ]
    out_shapes = []
    out_specs = []
    for k in KS:
        L = 2 * (k + 1) ** 2
        out_shapes.append(jax.ShapeDtypeStruct((BATCH, NE * L, VW), jnp.float32))
        out_specs.append(pl.BlockSpec((1, NPB * L, VW), lambda b, nb: (b, nb, 0)))
    scratch = [pltpu.VMEM((NE, 2 * (k + 1) ** 2, 2 * (k + 1) ** 2), jnp.float32)
               for k in KS]
    outs = pl.pallas_call(
        _tc_body,
        grid=(BATCH, NB),
        in_specs=in_specs,
        out_specs=out_specs,
        out_shape=out_shapes,
        scratch_shapes=scratch,
        compiler_params=pltpu.CompilerParams(vmem_limit_bytes=120 * 2 ** 20),
    )(*args)
    return {k: outs[j] for j, k in enumerate(KS)}


def _sc_assemble(bm1, bm3, bm5, g1, a1, g3, a3, g5, a5):
    mesh = plsc.VectorSubcoreMesh(core_axis_name="c", subcore_axis_name="s",
                                  num_cores=NC)
    acc_n = (TR + 1) * WROW
    scratch = [pltpu.VMEM((acc_n,), jnp.float32)]
    for k in KS:
        scratch += [pltpu.VMEM((CH[k], VW), jnp.float32),
                    pltpu.VMEM((CH[k], LP[k]), jnp.int32),
                    pltpu.VMEM((CH[k],), jnp.int32)]
    scratch.append(pltpu.SemaphoreType.DMA)

    @functools.partial(
        pl.kernel,
        out_type=jax.ShapeDtypeStruct((BATCH, NT, TR * WROW), jnp.float32),
        mesh=mesh,
        scratch_types=scratch,
        compiler_params=pltpu.CompilerParams(needs_layout_passes=False),
    )
    def body(bm1h, bm3h, bm5h, g1h, a1h, g3h, a3h, g5h, a5h, outh,
             acc, v1, av1, gv1, v3, av3, gv3, v5, av5, gv5, sem):
        wid = lax.axis_index("s") * NC + lax.axis_index("c")
        kparams = {1: (bm1h, g1h, a1h, v1, av1, gv1),
                   3: (bm3h, g3h, a3h, v3, av3, gv3),
                   5: (bm5h, g5h, a5h, v5, av5, gv5)}

        def task(i, _):
            tid = wid * (BATCH * NT // NW) + i
            b = tid // NT
            t = tid - b * NT

            def zero(jz, _):
                acc[pl.ds(jz * 16, 16)] = jnp.zeros((16,), jnp.float32)
                return 0

            lax.fori_loop(0, acc_n // 16, zero, 0)
            for k in KS:
                bmh, gh, ah, vv, av, gv = kparams[k]
                L = 2 * (k + 1) ** 2
                lp = LP[k]
                ch = CH[k]
                npad = _WL[k][2]
                boff = b * (NE * L)

                def chunk(c, _, gh=gh, ah=ah, bmh=bmh, vv=vv, av=av, gv=gv,
                          L=L, lp=lp, ch=ch, boff=boff):
                    pltpu.sync_copy(gh.at[t, pl.ds(c * ch, ch)], gv)

                    def addb(jb, _):
                        gv[pl.ds(jb * 16, 16)] = gv[pl.ds(jb * 16, 16)] + boff
                        return 0

                    lax.fori_loop(0, ch // 16, addb, 0)
                    pltpu.sync_copy(ah.at[t, pl.ds(c * ch, ch), :], av)
                    pltpu.async_copy(bmh.at[gv], vv, sem).wait()

                    def ent(iy, _):
                        allm = lax.iota(jnp.int32, 16) >= 0
                        for jj in range(lp // 16):
                            vals = vv[iy, pl.ds(jj * 16, 16)]
                            ad = av[iy, pl.ds(jj * 16, 16)]
                            plsc.addupdate_scatter(acc, [ad], vals, mask=allm)
                        return 0

                    lax.fori_loop(0, ch, ent, 0)
                    return 0

                lax.fori_loop(0, npad // ch, chunk, 0)
            pltpu.sync_copy(acc.at[pl.ds(0, TR * WROW)], outh.at[b, t])
            return 0

        lax.fori_loop(0, BATCH * NT // NW, task, 0)

    return body(bm1, bm3, bm5, g1, a1, g3, a3, g5, a5)


def _epi_body(kp_ref, zm_ref, out_ref):
    out_ref[0] = jnp.where(zm_ref[0], 0.0, kp_ref[0, :, 0:NDOF])


def _epilogue(kp, zero_map):
    return pl.pallas_call(
        _epi_body,
        grid=(BATCH,),
        in_specs=[pl.BlockSpec((1, NDOF, WROW), lambda b: (b, 0, 0)),
                  pl.BlockSpec((1, NDOF, NDOF), lambda b: (b, 0, 0))],
        out_specs=pl.BlockSpec((1, NDOF, NDOF), lambda b: (b, 0, 0)),
        out_shape=jax.ShapeDtypeStruct((BATCH, NDOF, NDOF), jnp.float32),
        compiler_params=pltpu.CompilerParams(vmem_limit_bytes=64 * 2 ** 20),
    )(kp, zero_map)


def _im2col(x, k):
    pad = (k - 1) // 2
    xp = jnp.pad(x, ((0, 0), (0, 0), (pad, pad), (pad, pad)))
    cols = []
    for c in range(2):
        for di in range(k):
            for dj in range(k):
                cols.append(xp[:, c, di:di + RES, dj:dj + RES].reshape(x.shape[0], -1))
    return jnp.stack(cols, axis=-1)


def kernel(x, zero_map, DBC, f, c11_w_1, c11_b_1, c12_w_1, c12_b_1, c13_w_1,
           c13_b_1, convt_w_1, c11_w_3, c11_b_3, c12_w_3, c12_b_3, c13_w_3,
           c13_b_3, convt_w_3, c11_w_5, c11_b_5, c12_w_5, c12_b_5, c13_w_5,
           c13_b_5, convt_w_5):
    raw = {1: (c11_w_1, c11_b_1, c12_w_1, c12_b_1, c13_w_1, c13_b_1, convt_w_1),
           3: (c11_w_3, c11_b_3, c12_w_3, c12_b_3, c13_w_3, c13_b_3, convt_w_3),
           5: (c11_w_5, c11_b_5, c12_w_5, c12_b_5, c13_w_5, c13_b_5, convt_w_5)}
    pats = {}
    weights = {}
    for k in KS:
        w1, b1, w2, b2, w3, b3, wm = raw[k]
        wi = w1.shape[0]
        pats[k] = _im2col(x, k)
        weights[k] = (w1.reshape(wi, 2 * k * k).T, b1.reshape(1, -1),
                      w2[:, :, 0, 0].T, b2.reshape(1, -1),
                      w3[:, :, 0, 0].T, b3.reshape(1, -1),
                      wm[:, 0, 0, :])
    bm = _run_tc(pats, weights)
    bmf = {k: bm[k].reshape(BATCH * NE * 2 * (k + 1) ** 2, VW) for k in KS}
    wl = {k: (jnp.asarray(_WL[k][0]),
              jnp.asarray(_WL[k][1].reshape(NT, _WL[k][2], LP[k])))
          for k in KS}
    shards = _sc_assemble(bmf[1], bmf[3], bmf[5],
                          wl[1][0], wl[1][1], wl[3][0], wl[3][1],
                          wl[5][0], wl[5][1])
    kp = shards.reshape(BATCH, NDOF, WROW)
    return _epilogue(kp, zero_map)
